# baseline (speedup 1.0000x reference)
import functools
import math

import jax
import jax.numpy as jnp
from jax import lax
from jax.experimental import pallas as pl
from jax.experimental.pallas import tpu as pltpu
from jax.experimental.pallas import tpu_sc as plsc

D_MODEL = 64
SCALE = math.sqrt(D_MODEL)

NUM_CORES = 2
NUM_SUBCORES = 16
NUM_WORKERS = NUM_CORES * NUM_SUBCORES

CHUNK = 160
CROWS = 320


def _compact_body(lut_hbm, out_hbm, i0, i1, c0, c1, isem, osem):
    wid = lax.axis_index("s") * NUM_CORES + lax.axis_index("c")
    v_total = lut_hbm.shape[0]
    n_chunks_tot = v_total // CROWS
    n_iter = -(-n_chunks_tot // NUM_WORKERS)

    ibuf = [i0, i1]
    cbuf = [c0, c1]

    def chunk_id(j):
        return jnp.minimum(wid + NUM_WORKERS * j, n_chunks_tot - 1)

    def start_in(j, s):
        off = pl.multiple_of(chunk_id(j) * CROWS, 8)
        pltpu.async_copy(lut_hbm.at[pl.ds(off, CROWS)], ibuf[s], isem)

    start_in(0, 0)
    start_in(1, 1)

    def half_step(j, s):
        pltpu.make_async_copy(
            lut_hbm.at[pl.ds(0, CROWS)], ibuf[s], isem).wait()

        @pl.when(j >= 2)
        def _():
            pltpu.make_async_copy(
                cbuf[s], out_hbm.at[pl.ds(0, CROWS // 2)], osem).wait()

        @plsc.parallel_loop(0, CROWS, unroll=8)
        def _(r):
            half = (r & 1) * D_MODEL
            for k in range(D_MODEL // 16):
                cbuf[s][r >> 1, pl.ds(half + 16 * k, 16)] = (
                    ibuf[s][r, pl.ds(16 * k, 16)])

        @pl.when(j + 2 < n_iter)
        def _():
            start_in(j + 2, s)

        ooff = pl.multiple_of(chunk_id(j) * (CROWS // 2), 8)
        pltpu.async_copy(cbuf[s], out_hbm.at[pl.ds(ooff, CROWS // 2)], osem)

    def pair_body(i, carry):
        half_step(2 * i, 0)
        half_step(2 * i + 1, 1)
        return carry

    lax.fori_loop(0, n_iter // 2, pair_body, 0)

    for s in range(2):
        pltpu.make_async_copy(
            cbuf[s], out_hbm.at[pl.ds(0, CROWS // 2)], osem).wait()


def _emb_body(x_hbm, lut_hbm, out_hbm, idx_all, p0, p1, g0, g1, o0, o1,
              gsem, osem):
    wid = lax.axis_index("s") * NUM_CORES + lax.axis_index("c")
    n_total = x_hbm.shape[0]
    b_per_w = n_total // NUM_WORKERS
    n_chunks = b_per_w // CHUNK
    base = wid * b_per_w

    pidx = [p0, p1]
    grows = [g0, g1]
    orows = [o0, o1]

    pltpu.sync_copy(x_hbm.at[pl.ds(base, b_per_w)],
                    idx_all.at[pl.ds(0, b_per_w)])

    def fill_pidx(g, s):
        def vec(i, c):
            v = idx_all[pl.ds(g * CHUNK + i * 16, 16)]
            pidx[s][pl.ds(i * 16, 16)] = jax.lax.shift_right_logical(v, 1)
            return c
        lax.fori_loop(0, CHUNK // 16, vec, 0, unroll=4)

    fill_pidx(0, 0)
    pltpu.async_copy(lut_hbm.at[pidx[0]], grows[0], gsem)
    fill_pidx(1, 1)
    pltpu.async_copy(lut_hbm.at[pidx[1]], grows[1], gsem)

    def half_step(g, s):
        pltpu.make_async_copy(lut_hbm.at[pidx[s]], grows[s], gsem).wait()

        @pl.when(g >= 2)
        def _():
            pltpu.make_async_copy(
                orows[s], out_hbm.at[pl.ds(base, CHUNK)], osem).wait()

        @plsc.parallel_loop(0, CHUNK, unroll=8)
        def _(r):
            v = idx_all[pl.ds(g * CHUNK + r, 16)]
            par = v[0] & 1
            doff = par * D_MODEL
            for k in range(D_MODEL // 16):
                orows[s][r, pl.ds(16 * k, 16)] = (
                    grows[s][r, pl.ds(doff + 16 * k, 16)] * SCALE)

        @pl.when(g + 2 < n_chunks)
        def _():
            fill_pidx(g + 2, s)
            pltpu.async_copy(lut_hbm.at[pidx[s]], grows[s], gsem)

        pltpu.async_copy(orows[s], out_hbm.at[pl.ds(base + g * CHUNK, CHUNK)],
                         osem)

    def pair_body(i, carry):
        half_step(2 * i, 0)
        half_step(2 * i + 1, 1)
        return carry

    lax.fori_loop(0, n_chunks // 2, pair_body, 0)

    for s in range(2):
        pltpu.make_async_copy(
            orows[s], out_hbm.at[pl.ds(base, CHUNK)], osem).wait()


def kernel(x, lut):
    b, s = x.shape
    n = b * s
    xf = x.reshape(n).astype(jnp.int32)
    v_total, d = lut.shape

    mesh = plsc.VectorSubcoreMesh(
        core_axis_name="c", subcore_axis_name="s",
        num_cores=NUM_CORES, num_subcores=NUM_SUBCORES,
    )
    params = pltpu.CompilerParams(use_tc_tiling_on_sc=True)

    compact_call = pl.kernel(
        _compact_body,
        out_type=jax.ShapeDtypeStruct((v_total // 2, 2 * d), jnp.float32),
        mesh=mesh,
        scratch_types=[
            pltpu.VMEM((CROWS, d), jnp.float32),
            pltpu.VMEM((CROWS, d), jnp.float32),
            pltpu.VMEM((CROWS // 2, 2 * d), jnp.float32),
            pltpu.VMEM((CROWS // 2, 2 * d), jnp.float32),
            pltpu.SemaphoreType.DMA,
            pltpu.SemaphoreType.DMA,
        ],
        compiler_params=params,
    )
    lut2 = compact_call(lut)

    emb_call = pl.kernel(
        _emb_body,
        out_type=jax.ShapeDtypeStruct((n, D_MODEL), jnp.float32),
        mesh=mesh,
        scratch_types=[
            pltpu.VMEM((n // NUM_WORKERS + 16,), jnp.int32),
            pltpu.VMEM((CHUNK,), jnp.int32),
            pltpu.VMEM((CHUNK,), jnp.int32),
            pltpu.VMEM((CHUNK, 2 * D_MODEL), jnp.float32),
            pltpu.VMEM((CHUNK, 2 * D_MODEL), jnp.float32),
            pltpu.VMEM((CHUNK, D_MODEL), jnp.float32),
            pltpu.VMEM((CHUNK, D_MODEL), jnp.float32),
            pltpu.SemaphoreType.DMA,
            pltpu.SemaphoreType.DMA,
        ],
        compiler_params=params,
    )
    out = emb_call(xf, lut2)
    return out.reshape(b, s, D_MODEL)


# speedup vs baseline: 1.0088x; 1.0088x over previous
import functools
import math

import jax
import jax.numpy as jnp
from jax import lax
from jax.experimental import pallas as pl
from jax.experimental.pallas import tpu as pltpu
from jax.experimental.pallas import tpu_sc as plsc

D_MODEL = 64
SCALE = math.sqrt(D_MODEL)

NUM_CORES = 2
NUM_SUBCORES = 16
NUM_WORKERS = NUM_CORES * NUM_SUBCORES

CHUNK = 160


def _emb_body(x_hbm, lut_hbm, out_hbm, idx_all, p0, p1, g0, g1, o0, o1,
              gsem, osem):
    wid = lax.axis_index("s") * NUM_CORES + lax.axis_index("c")
    n_total = x_hbm.shape[0]
    b_per_w = n_total // NUM_WORKERS
    n_chunks = b_per_w // CHUNK
    base = wid * b_per_w

    pidx = [p0, p1]
    grows = [g0, g1]
    orows = [o0, o1]

    pltpu.sync_copy(x_hbm.at[pl.ds(base, b_per_w)],
                    idx_all.at[pl.ds(0, b_per_w)])

    def fill_pidx(g, s):
        def vec(i, c):
            v = idx_all[pl.ds(g * CHUNK + i * 16, 16)]
            pidx[s][pl.ds(i * 16, 16)] = jax.lax.shift_right_logical(v, 1)
            return c
        lax.fori_loop(0, CHUNK // 16, vec, 0, unroll=4)

    fill_pidx(0, 0)
    pltpu.async_copy(lut_hbm.at[pidx[0]], grows[0], gsem)
    fill_pidx(1, 1)
    pltpu.async_copy(lut_hbm.at[pidx[1]], grows[1], gsem)

    def half_step(g, s):
        pltpu.make_async_copy(lut_hbm.at[pidx[s]], grows[s], gsem).wait()

        @pl.when(g >= 2)
        def _():
            pltpu.make_async_copy(
                orows[s], out_hbm.at[pl.ds(base, CHUNK)], osem).wait()

        @plsc.parallel_loop(0, CHUNK, unroll=8)
        def _(r):
            v = idx_all[pl.ds(g * CHUNK + r, 16)]
            par = v[0] & 1
            doff = par * D_MODEL
            for k in range(D_MODEL // 16):
                orows[s][r, pl.ds(16 * k, 16)] = (
                    grows[s][r, pl.ds(doff + 16 * k, 16)] * SCALE)

        @pl.when(g + 2 < n_chunks)
        def _():
            fill_pidx(g + 2, s)
            pltpu.async_copy(lut_hbm.at[pidx[s]], grows[s], gsem)

        pltpu.async_copy(orows[s], out_hbm.at[pl.ds(base + g * CHUNK, CHUNK)],
                         osem)

    def pair_body(i, carry):
        half_step(2 * i, 0)
        half_step(2 * i + 1, 1)
        return carry

    lax.fori_loop(0, n_chunks // 2, pair_body, 0)

    for s in range(2):
        pltpu.make_async_copy(
            orows[s], out_hbm.at[pl.ds(base, CHUNK)], osem).wait()


def kernel(x, lut):
    b, s = x.shape
    n = b * s
    xf = x.reshape(n).astype(jnp.int32)
    lut2 = lut.reshape(lut.shape[0] // 2, 2 * lut.shape[1])

    emb_call = pl.kernel(
        _emb_body,
        out_type=jax.ShapeDtypeStruct((n, D_MODEL), jnp.float32),
        mesh=plsc.VectorSubcoreMesh(
            core_axis_name="c", subcore_axis_name="s",
            num_cores=NUM_CORES, num_subcores=NUM_SUBCORES,
        ),
        scratch_types=[
            pltpu.VMEM((n // NUM_WORKERS + 16,), jnp.int32),
            pltpu.VMEM((CHUNK,), jnp.int32),
            pltpu.VMEM((CHUNK,), jnp.int32),
            pltpu.VMEM((CHUNK, 2 * D_MODEL), jnp.float32),
            pltpu.VMEM((CHUNK, 2 * D_MODEL), jnp.float32),
            pltpu.VMEM((CHUNK, D_MODEL), jnp.float32),
            pltpu.VMEM((CHUNK, D_MODEL), jnp.float32),
            pltpu.SemaphoreType.DMA,
            pltpu.SemaphoreType.DMA,
        ],
        compiler_params=pltpu.CompilerParams(use_tc_tiling_on_sc=True),
    )
    out = emb_call(xf, lut2)
    return out.reshape(b, s, D_MODEL)
